# Initial kernel scaffold; baseline (speedup 1.0000x reference)
import jax
import jax.numpy as jnp
from jax.experimental import pallas as pl


def kernel(xa, edge_index, starts, first_message, enc_W, enc_b, ns_W, ns_b, nm_W, nm_b, act_W, act_b, dec_W, dec_b):
    raise NotImplementedError("write your pallas kernel here")


# speedup vs baseline: 662.7345x; 662.7345x over previous
import jax
import jax.numpy as jnp
from jax import lax
from jax.experimental import pallas as pl
from jax.experimental.pallas import tpu as pltpu
from jax.experimental.pallas import tpu_sc as plsc

N = 256
E = 2048
HID = 128
MSG = 32
CAT = HID + MSG
OUT_F = 32
NPRED = 2
BUDGET = N * 10
QCAP = BUDGET + N + 16
NBR_SZ = 2 * E + 16
RP_SZ = 272
MS_ROWS = N + BUDGET
NSUB = 16
ZROWS = (MS_ROWS - N) // NSUB


def _enc_body(x_ref, w_ref, b_ref, o_ref):
    o_ref[...] = (
        jnp.dot(x_ref[...], w_ref[...], preferred_element_type=jnp.float32)
        + b_ref[...]
    )


def _dec_body(ff_ref, w_ref, b_ref, o_ref):
    x = ff_ref[...]
    for p in range(NPRED):
        o = jnp.dot(x, w_ref[p], preferred_element_type=jnp.float32) + b_ref[p]
        m = jnp.max(o, axis=-1, keepdims=True)
        e = o - m
        o_ref[p] = e - jnp.log(jnp.sum(jnp.exp(e), axis=-1, keepdims=True))


def _sc_body(pred0_hbm, fm_hbm, wns_hbm, wnm_hbm, wactb_hbm, nsb_hbm, nmb_hbm,
             nbr_hbm, rowptr_hbm, q0_hbm, scal_hbm, out_hbm,
             pred_v, ff_v, wnsl_v, wnm_v, wactb_v, nsb_v, nmb_v, nbr_v,
             rowptr_v, qpk_v, cat_v, msgbuf_v, nsg_v, nswr_v, nmp_v, idx1_v,
             zbuf_v, scal_v, ta_s, msg_sp, ns_sh):
    wid = lax.axis_index("s")
    zf = jnp.zeros((16,), jnp.float32)
    lane0 = lax.iota(jnp.int32, 16) == 0

    def spl(i):
        return jnp.full((16,), i, jnp.int32)

    def sload(ref, i):
        return jnp.max(plsc.load_gather(ref, [spl(i)]))

    pltpu.sync_copy(pred0_hbm, pred_v)
    pltpu.sync_copy(wnm_hbm, wnm_v)
    pltpu.sync_copy(wactb_hbm, wactb_v)
    pltpu.sync_copy(nsb_hbm, nsb_v)
    pltpu.sync_copy(nmb_hbm, nmb_v)
    pltpu.sync_copy(nbr_hbm, nbr_v)
    pltpu.sync_copy(rowptr_hbm, rowptr_v)
    pltpu.sync_copy(q0_hbm, qpk_v)
    pltpu.sync_copy(scal_hbm, scal_v)

    @pl.when(wid < 8)
    def _stage_ns_slice():
        pltpu.sync_copy(wns_hbm.at[pl.ds(wid * (CAT * 16), CAT * 16)], wnsl_v)

    @pl.when(wid == 0)
    def _stage_fm():
        pltpu.sync_copy(fm_hbm, msg_sp.at[pl.ds(0, N)])

    def zff(i, _):
        ff_v[pl.ds(i * 16, 16)] = zf
        return 0

    lax.fori_loop(0, N * HID // 16, zff, 0)

    def zta(i, _):
        ta_s[i] = jnp.float32(0.0)
        return 0

    lax.fori_loop(0, N, zta, 0)

    def zzb(i, _):
        zbuf_v[i, pl.ds(0, 16)] = zf
        zbuf_v[i, pl.ds(16, 16)] = zf
        return 0

    lax.fori_loop(0, ZROWS, zzb, 0)
    pltpu.sync_copy(zbuf_v, msg_sp.at[pl.ds(N + wid * ZROWS, ZROWS)])

    plsc.subcore_barrier()

    tail0 = sload(scal_v, 0)
    actb = sload(wactb_v, CAT)

    def cond(cy):
        head, tail, _ = cy
        return jnp.logical_and(head < tail, head < BUDGET)

    def body(cy):
        head, tail, mcount = cy
        pk = sload(qpk_v, head)
        node = pk & jnp.int32(0xFFFF)
        ta_n = ta_s[node]
        do = jnp.logical_not(ta_n > 1.0 - 1e-7)

        def active(_):
            mid = lax.shift_right_logical(pk, jnp.int32(16))
            base = sload(rowptr_v, node)
            dn = sload(rowptr_v, node + 1) - base
            pltpu.sync_copy(msg_sp.at[mid], msgbuf_v)
            for j in range(HID // 16):
                cat_v[pl.ds(j * 16, 16)] = pred_v[pl.ds(node * HID + j * 16, 16)]
            for j in range(MSG // 16):
                cat_v[pl.ds(HID + j * 16, 16)] = msgbuf_v[pl.ds(j * 16, 16)]

            acc = zf
            for j in range(CAT // 16):
                acc = acc + cat_v[pl.ds(j * 16, 16)] * wactb_v[pl.ds(j * 16, 16)]
            z = jnp.sum(acc) + actb
            sig = 1.0 / (1.0 + jnp.exp(jnp.full((16,), -z)))
            cand = jnp.max(sig)
            over = (ta_n + cand) > 1.0
            new_act = jnp.where(over, 1.0 - ta_n, cand)

            @pl.when(wid < 8)
            def _ns_slice():
                def ns_k(kk, a):
                    k = kk * 4
                    for u in range(4):
                        ck = plsc.load_gather(cat_v, [spl(k + u)])
                        a = a + ck * wnsl_v[pl.ds((k + u) * 16, 16)]
                    return a

                a = lax.fori_loop(0, CAT // 4, ns_k, zf)
                nsj = jnp.maximum(a + nsb_v[pl.ds(wid * 16, 16)], 0.0)
                nswr_v[pl.ds(0, 16)] = nsj
                pltpu.sync_copy(nswr_v, ns_sh.at[pl.ds(wid * 16, 16)])
                nmp_v[0, pl.ds(0, 16)] = zf
                nmp_v[0, pl.ds(16, 16)] = zf

            @pl.when(wid >= 8)
            def _nm_msg_part():
                kb = HID + (wid - 8) * 4
                a0 = zf
                a1 = zf
                for u in range(4):
                    ck = plsc.load_gather(cat_v, [spl(kb + u)])
                    a0 = a0 + ck * wnm_v[pl.ds((kb + u) * MSG, 16)]
                    a1 = a1 + ck * wnm_v[pl.ds((kb + u) * MSG + 16, 16)]
                nmp_v[0, pl.ds(0, 16)] = a0
                nmp_v[0, pl.ds(16, 16)] = a1

                @pl.when(wid == 8)
                def _nm_bias():
                    nmp_v[0, pl.ds(0, 16)] = nmp_v[0, pl.ds(0, 16)] + nmb_v[pl.ds(0, 16)]
                    nmp_v[0, pl.ds(16, 16)] = nmp_v[0, pl.ds(16, 16)] + nmb_v[pl.ds(16, 16)]

            plsc.subcore_barrier()

            pltpu.sync_copy(ns_sh, nsg_v)
            navec = jnp.full((16,), new_act)
            for j in range(HID // 16):
                nsj = nsg_v[pl.ds(j * 16, 16)]
                off = node * HID + j * 16
                pred_v[pl.ds(off, 16)] = nsj
                ff_v[pl.ds(off, 16)] = ff_v[pl.ds(off, 16)] + nsj * navec

            a0 = nmp_v[0, pl.ds(0, 16)]
            a1 = nmp_v[0, pl.ds(16, 16)]
            for u in range(8):
                k = wid * 8 + u
                ck = plsc.load_gather(nsg_v, [spl(k)])
                a0 = a0 + ck * wnm_v[pl.ds(k * MSG, 16)]
                a1 = a1 + ck * wnm_v[pl.ds(k * MSG + 16, 16)]
            nmp_v[0, pl.ds(0, 16)] = a0
            nmp_v[0, pl.ds(16, 16)] = a1
            plsc.store_scatter(idx1_v, [spl(0)], spl(mcount), mask=lane0)
            pltpu.sync_copy(nmp_v, msg_sp.at[idx1_v], add=True)

            ta_s[node] = ta_n + new_act

            @pl.when(tail < BUDGET)
            def _push():
                mcv = spl(lax.shift_left(mcount, jnp.int32(16)))

                def pb(j, _):
                    qpk_v[pl.ds(tail + j * 16, 16)] = (
                        nbr_v[pl.ds(base + j * 16, 16)] | mcv
                    )
                    return 0

                lax.fori_loop(0, (dn + 15) // 16, pb, 0)

            plsc.subcore_barrier()
            return dn

        dni = lax.cond(do, active, lambda _: jnp.int32(0), 0)
        mci = jnp.where(do, jnp.int32(1), jnp.int32(0))
        return (head + jnp.int32(1), tail + dni, mcount + mci)

    lax.while_loop(cond, body, (jnp.int32(0), tail0, jnp.int32(N)))

    @pl.when(wid == 0)
    def _out():
        pltpu.sync_copy(ff_v, out_hbm)


def _sc_loop(encoded, first_message, ns_W, ns_b, nm_W, nm_b, act_W, act_b,
             nbr_flat, rowptr_pad, q0, scal):
    sc = pl.kernel(
        _sc_body,
        out_type=jax.ShapeDtypeStruct((N * HID,), jnp.float32),
        mesh=plsc.VectorSubcoreMesh(core_axis_name="c", subcore_axis_name="s",
                                    num_cores=1),
        compiler_params=pltpu.CompilerParams(needs_layout_passes=False),
        scratch_types=[
            pltpu.VMEM((N * HID,), jnp.float32),
            pltpu.VMEM((N * HID,), jnp.float32),
            pltpu.VMEM((CAT * 16,), jnp.float32),
            pltpu.VMEM((CAT * MSG,), jnp.float32),
            pltpu.VMEM((176,), jnp.float32),
            pltpu.VMEM((HID,), jnp.float32),
            pltpu.VMEM((MSG,), jnp.float32),
            pltpu.VMEM((NBR_SZ,), jnp.int32),
            pltpu.VMEM((RP_SZ,), jnp.int32),
            pltpu.VMEM((QCAP,), jnp.int32),
            pltpu.VMEM((CAT,), jnp.float32),
            pltpu.VMEM((MSG,), jnp.float32),
            pltpu.VMEM((HID,), jnp.float32),
            pltpu.VMEM((16,), jnp.float32),
            pltpu.VMEM((1, MSG), jnp.float32),
            pltpu.VMEM((1,), jnp.int32),
            pltpu.VMEM((ZROWS, MSG), jnp.float32),
            pltpu.VMEM((16,), jnp.int32),
            pltpu.SMEM((N,), jnp.float32),
            pltpu.VMEM_SHARED((MS_ROWS, MSG), jnp.float32),
            pltpu.VMEM_SHARED((HID,), jnp.float32),
        ],
    )
    wactb = jnp.concatenate([act_W[:, 0], act_b, jnp.zeros((15,), jnp.float32)])
    wns_sliced = ns_W.reshape(CAT, 8, 16).transpose(1, 0, 2).reshape(-1)
    return sc(encoded.reshape(-1), first_message,
              wns_sliced, nm_W.reshape(-1), wactb,
              ns_b, nm_b, nbr_flat, rowptr_pad, q0, scal)


def kernel(xa, edge_index, starts, first_message, enc_W, enc_b, ns_W, ns_b,
           nm_W, nm_b, act_W, act_b, dec_W, dec_b):
    src, dst = edge_index[0], edge_index[1]
    adj = jnp.zeros((N, N), jnp.bool_).at[src, dst].set(True).at[dst, src].set(True)
    deg = jnp.sum(adj, axis=1, dtype=jnp.int32)
    rowptr = jnp.concatenate([jnp.zeros((1,), jnp.int32),
                              jnp.cumsum(deg, dtype=jnp.int32)])
    order = jnp.argsort(jnp.logical_not(adj), axis=1, stable=True).astype(jnp.int32)
    col = jnp.arange(N, dtype=jnp.int32)
    pos = rowptr[:N, None] + col[None, :]
    valid = col[None, :] < deg[:, None]
    nbr_flat = (jnp.zeros((NBR_SZ,), jnp.int32)
                .at[jnp.where(valid, pos, NBR_SZ)].set(order, mode='drop'))
    rowptr_pad = jnp.zeros((RP_SZ,), jnp.int32).at[:257].set(rowptr)

    start_mask = starts != 0
    offs0 = jnp.cumsum(start_mask.astype(jnp.int32)) - 1
    pos0 = jnp.where(start_mask, offs0, QCAP)
    q0 = (jnp.zeros((QCAP,), jnp.int32)
          .at[pos0].set(col | (col << 16), mode='drop'))
    tail0 = jnp.sum(start_mask).astype(jnp.int32)
    scal = jnp.zeros((16,), jnp.int32).at[0].set(tail0)

    encoded = pl.pallas_call(
        _enc_body,
        out_shape=jax.ShapeDtypeStruct((N, HID), jnp.float32),
    )(xa, enc_W, enc_b.reshape(1, HID))

    ff = _sc_loop(encoded, first_message, ns_W, ns_b, nm_W, nm_b, act_W, act_b,
                  nbr_flat, rowptr_pad, q0, scal).reshape(N, HID)

    return pl.pallas_call(
        _dec_body,
        out_shape=jax.ShapeDtypeStruct((NPRED, N, OUT_F), jnp.float32),
    )(ff, dec_W, dec_b)
